# baseline (speedup 1.0000x reference)
import functools

import jax
import jax.numpy as jnp
from jax import lax
from jax.experimental import pallas as pl
from jax.experimental.pallas import tpu as pltpu
from jax.experimental.pallas import tpu_sc as plsc

V = 100000
D = 128
L = 200
R = V // 8


def _sc_body(ctx_hbm, tab_hbm, out_hbm, idx_v, rows_v, vsum_v, sem):
    cid = lax.axis_index("c")
    sid = lax.axis_index("s")

    @pl.when(jnp.logical_and(cid == 0, sid == 0))
    def _():
        pltpu.sync_copy(ctx_hbm, idx_v)
        cp0 = pltpu.async_copy(
            tab_hbm.at[idx_v.at[pl.ds(0, 128)]], rows_v.at[pl.ds(0, 128)], sem
        )
        cp1 = pltpu.async_copy(
            tab_hbm.at[idx_v.at[pl.ds(128, L - 128)]],
            rows_v.at[pl.ds(128, L - 128)],
            sem,
        )
        cp0.wait()
        cp1.wait()

        def sum_body(i, acc):
            return tuple(
                acc[j] + rows_v[i, pl.ds(j * 16, 16)] for j in range(D // 16)
            )

        acc = lax.fori_loop(
            0,
            L,
            sum_body,
            tuple(jnp.zeros((16,), jnp.float32) for _ in range(D // 16)),
        )
        scale = jnp.float32(1.0 / L)
        for j in range(D // 16):
            vsum_v[pl.ds(j * 16, 16)] = acc[j] * scale
        pltpu.sync_copy(vsum_v, out_hbm)


@functools.cache
def _sc_gather_mean():
    return pl.kernel(
        _sc_body,
        out_type=jax.ShapeDtypeStruct((D,), jnp.float32),
        mesh=plsc.VectorSubcoreMesh(core_axis_name="c", subcore_axis_name="s"),
        scratch_types=[
            pltpu.VMEM((L,), jnp.int32),
            pltpu.VMEM((L, D), jnp.float32),
            pltpu.VMEM((D,), jnp.float32),
            pltpu.SemaphoreType.DMA,
        ],
    )


B3 = 400
NS = 4
NB = -(-R // B3)
NH = NB // NS
RP = NB * B3
assert NB % NS == 0


def _tc_body(*refs):
    v_ref = refs[0]
    w_refs = refs[1 : 1 + NS]
    b_refs = refs[1 + NS : 1 + 2 * NS]
    out_ref = refs[1 + 2 * NS]
    acc_ref = refs[2 + 2 * NS]
    i = pl.program_id(0)

    @pl.when(i == 0)
    def _():
        acc_ref[0] = -jnp.inf
        acc_ref[1] = 0.0

    vb = v_ref[...].reshape(1, 1, D)
    los, valids, bmaxs = [], [], []
    for k in range(NS):
        base = (i + k * NH) * B3
        lo = jnp.sum(w_refs[k][...] * vb, axis=2) + b_refs[k][...]
        out_ref[pl.ds(base, B3), :] = lo
        gs = lax.broadcasted_iota(jnp.int32, (B3, 8), 0) + base
        valid = gs < R
        los.append(lo)
        valids.append(valid)
        bmaxs.append(jnp.max(jnp.where(valid, lo, -jnp.inf)))

    m_old = acc_ref[0]
    s_old = acc_ref[1]
    m_new = m_old
    for bm in bmaxs:
        m_new = jnp.maximum(m_new, bm)
    s_new = s_old * jnp.exp(m_old - m_new)
    for lo, valid in zip(los, valids):
        s_new = s_new + jnp.sum(jnp.where(valid, jnp.exp(lo - m_new), 0.0))
    acc_ref[0] = m_new
    acc_ref[1] = s_new

    @pl.when(i == NH - 1)
    def _():
        lse = m_new + jnp.log(s_new)
        out_ref[...] = out_ref[...] - lse


def _mk_w_spec(k):
    return pl.BlockSpec((B3, 8, D), lambda i, k=k: (i + k * NH, 0, 0))


def _mk_b_spec(k):
    return pl.BlockSpec((B3, 8), lambda i, k=k: (i + k * NH, 0))


_tc_matvec_lse = pl.pallas_call(
    _tc_body,
    grid=(NH,),
    in_specs=(
        [pl.BlockSpec((1, D), lambda i: (0, 0))]
        + [_mk_w_spec(k) for k in range(NS)]
        + [_mk_b_spec(k) for k in range(NS)]
    ),
    out_specs=pl.BlockSpec((RP, 8), lambda i: (0, 0)),
    out_shape=jax.ShapeDtypeStruct((RP, 8), jnp.float32),
    scratch_shapes=[
        pltpu.SMEM((2,), jnp.float32),
    ],
    compiler_params=pltpu.CompilerParams(
        dimension_semantics=("arbitrary",)
    ),
)


def kernel(context, emb_table, W, b):
    context = context.astype(jnp.int32)
    v = _sc_gather_mean()(context, emb_table)
    w3 = W.reshape(R, 8, D)
    b3 = b.reshape(R, 8)
    padded = _tc_matvec_lse(
        v.reshape(1, D), *([w3] * NS), *([b3] * NS)
    )
    return padded[:R, :].reshape(1, V)


# speedup vs baseline: 2.0171x; 2.0171x over previous
import functools

import jax
import jax.numpy as jnp
from jax import lax
from jax.experimental import pallas as pl
from jax.experimental.pallas import tpu as pltpu
from jax.experimental.pallas import tpu_sc as plsc

V = 100000
D = 128
L = 200


def _sc_body(ctx_hbm, tab_hbm, out_hbm, idx_v, rows_v, vsum_v, sem):
    cid = lax.axis_index("c")
    sid = lax.axis_index("s")

    @pl.when(jnp.logical_and(cid == 0, sid == 0))
    def _():
        pltpu.sync_copy(ctx_hbm, idx_v)
        cp0 = pltpu.async_copy(
            tab_hbm.at[idx_v.at[pl.ds(0, 128)]], rows_v.at[pl.ds(0, 128)], sem
        )
        cp1 = pltpu.async_copy(
            tab_hbm.at[idx_v.at[pl.ds(128, L - 128)]],
            rows_v.at[pl.ds(128, L - 128)],
            sem,
        )
        cp0.wait()
        cp1.wait()

        def sum_body(i, acc):
            return tuple(
                acc[j] + rows_v[i, pl.ds(j * 16, 16)] for j in range(D // 16)
            )

        acc = lax.fori_loop(
            0,
            L,
            sum_body,
            tuple(jnp.zeros((16,), jnp.float32) for _ in range(D // 16)),
        )
        scale = jnp.float32(1.0 / L)
        for j in range(D // 16):
            vsum_v[pl.ds(j * 16, 16)] = acc[j] * scale
        pltpu.sync_copy(vsum_v, out_hbm)


@functools.cache
def _sc_gather_mean():
    return pl.kernel(
        _sc_body,
        out_type=jax.ShapeDtypeStruct((D,), jnp.float32),
        mesh=plsc.VectorSubcoreMesh(core_axis_name="c", subcore_axis_name="s"),
        scratch_types=[
            pltpu.VMEM((L,), jnp.int32),
            pltpu.VMEM((L, D), jnp.float32),
            pltpu.VMEM((D,), jnp.float32),
            pltpu.SemaphoreType.DMA,
        ],
    )

BLK = 3200
NS = 4
NB = -(-V // BLK)
NH = NB // NS
assert NB % NS == 0


def _tc1_body(*refs):
    v_ref = refs[0]
    w_refs = refs[1 : 1 + NS]
    b_refs = refs[1 + NS : 1 + 2 * NS]
    out_ref = refs[1 + 2 * NS]
    acc_ref = refs[2 + 2 * NS]
    i = pl.program_id(0)

    @pl.when(i == 0)
    def _():
        acc_ref[0] = -jnp.inf
        acc_ref[1] = 0.0

    vm = v_ref[...]
    dn = (((1,), (0,)), ((), ()))
    los, valids, bmaxs = [], [], []
    for k in range(NS):
        lo_col = lax.dot_general(
            w_refs[k][...], vm, dn, preferred_element_type=jnp.float32
        )
        lo = lax.transpose(lo_col, (1, 0))[0:1, :]
        lo = lo + b_refs[k][...]
        out_ref[:, pl.ds((i + k * NH) * BLK, BLK)] = lo
        pos = lax.broadcasted_iota(jnp.int32, (1, BLK), 1) + (i + k * NH) * BLK
        valid = pos < V
        los.append(lo)
        valids.append(valid)
        bmaxs.append(jnp.max(jnp.where(valid, lo, -jnp.inf)))

    m_old = acc_ref[0]
    s_old = acc_ref[1]
    m_new = m_old
    for bm in bmaxs:
        m_new = jnp.maximum(m_new, bm)
    s_new = s_old * jnp.exp(m_old - m_new)
    for lo, valid in zip(los, valids):
        s_new = s_new + jnp.sum(jnp.where(valid, jnp.exp(lo - m_new), 0.0))
    acc_ref[0] = m_new
    acc_ref[1] = s_new

    @pl.when(i == NH - 1)
    def _():
        lse = m_new + jnp.log(s_new)
        out_ref[...] = out_ref[...] - lse


def _mk_w_spec(k):
    return pl.BlockSpec((BLK, D), lambda i, k=k: (i + k * NH, 0))


def _mk_b_spec(k):
    return pl.BlockSpec((1, BLK), lambda i, k=k: (0, i + k * NH))


_tc_matvec_lse = pl.pallas_call(
    _tc1_body,
    grid=(NH,),
    in_specs=(
        [pl.BlockSpec((D, 8), lambda i: (0, 0))]
        + [_mk_w_spec(k) for k in range(NS)]
        + [_mk_b_spec(k) for k in range(NS)]
    ),
    out_specs=pl.BlockSpec((1, NB * BLK), lambda i: (0, 0)),
    out_shape=jax.ShapeDtypeStruct((1, NB * BLK), jnp.float32),
    scratch_shapes=[
        pltpu.SMEM((2,), jnp.float32),
    ],
    compiler_params=pltpu.CompilerParams(
        dimension_semantics=("arbitrary",)
    ),
)


def kernel(context, emb_table, W, b):
    context = context.astype(jnp.int32)
    v = _sc_gather_mean()(context, emb_table)
    b2 = b.reshape(1, V)
    vm = jnp.broadcast_to(v.reshape(D, 1), (D, 8))
    padded = _tc_matvec_lse(vm, *([W] * NS), *([b2] * NS))
    return padded[:, :V]


# speedup vs baseline: 2.1702x; 1.0759x over previous
import functools

import jax
import jax.numpy as jnp
from jax import lax
from jax.experimental import pallas as pl
from jax.experimental.pallas import tpu as pltpu
from jax.experimental.pallas import tpu_sc as plsc

V = 100000
D = 128
L = 200


def _sc_body(ctx_hbm, tab_hbm, out_hbm, idx_v, rows_v, vsum_v, sem):
    cid = lax.axis_index("c")
    sid = lax.axis_index("s")

    @pl.when(jnp.logical_and(cid == 0, sid == 0))
    def _():
        pltpu.sync_copy(ctx_hbm, idx_v)
        cp0 = pltpu.async_copy(
            tab_hbm.at[idx_v.at[pl.ds(0, 128)]], rows_v.at[pl.ds(0, 128)], sem
        )
        cp1 = pltpu.async_copy(
            tab_hbm.at[idx_v.at[pl.ds(128, L - 128)]],
            rows_v.at[pl.ds(128, L - 128)],
            sem,
        )
        cp0.wait()
        cp1.wait()

        def sum_body(i, acc):
            return tuple(
                acc[j] + rows_v[i, pl.ds(j * 16, 16)] for j in range(D // 16)
            )

        acc = lax.fori_loop(
            0,
            L,
            sum_body,
            tuple(jnp.zeros((16,), jnp.float32) for _ in range(D // 16)),
        )
        scale = jnp.float32(1.0 / L)
        for j in range(D // 16):
            vsum_v[0, pl.ds(j * 16, 16)] = acc[j] * scale
        pltpu.sync_copy(vsum_v, out_hbm)


@functools.cache
def _sc_gather_mean():
    return pl.kernel(
        _sc_body,
        out_type=jax.ShapeDtypeStruct((1, D), jnp.float32),
        mesh=plsc.VectorSubcoreMesh(core_axis_name="c", subcore_axis_name="s"),
        scratch_types=[
            pltpu.VMEM((L,), jnp.int32),
            pltpu.VMEM((L, D), jnp.float32),
            pltpu.VMEM((1, D), jnp.float32),
            pltpu.SemaphoreType.DMA,
        ],
    )

BLK = 3200
NS = 4
NB = -(-V // BLK)
NH = NB // NS
VT = V - (NB - 1) * BLK
assert NB % NS == 0


def _tc1_body(*refs):
    v_ref = refs[0]
    w_refs = refs[1 : 1 + NS]
    b_refs = refs[1 + NS : 1 + 2 * NS]
    out_ref = refs[1 + 2 * NS]
    acc_ref = refs[2 + 2 * NS]
    i = pl.program_id(0)

    @pl.when(i == 0)
    def _():
        acc_ref[0] = -jnp.inf
        acc_ref[1] = 0.0

    vm = lax.transpose(jnp.broadcast_to(v_ref[...], (8, D)), (1, 0))
    dn = (((1,), (0,)), ((), ()))
    los, valids, bmaxs = [], [], []
    for k in range(NS):
        lo_col = lax.dot_general(
            w_refs[k][...], vm, dn, preferred_element_type=jnp.float32
        )
        lo = lax.transpose(lo_col, (1, 0))[0:1, :]
        lo = lo + b_refs[k][...]
        base = (i + k * NH) * BLK
        if k == NS - 1:
            @pl.when(i < NH - 1)
            def _(lo=lo, base=base):
                out_ref[:, pl.ds(base, BLK)] = lo

            @pl.when(i == NH - 1)
            def _(lo=lo, base=base):
                out_ref[:, pl.ds(base, VT)] = lo[:, :VT]
        else:
            out_ref[:, pl.ds(base, BLK)] = lo
        pos = lax.broadcasted_iota(jnp.int32, (1, BLK), 1) + base
        valid = pos < V
        los.append(lo)
        valids.append(valid)
        bmaxs.append(jnp.max(jnp.where(valid, lo, -jnp.inf)))

    m_old = acc_ref[0]
    s_old = acc_ref[1]
    m_new = m_old
    for bm in bmaxs:
        m_new = jnp.maximum(m_new, bm)
    s_new = s_old * jnp.exp(m_old - m_new)
    for lo, valid in zip(los, valids):
        s_new = s_new + jnp.sum(jnp.where(valid, jnp.exp(lo - m_new), 0.0))
    acc_ref[0] = m_new
    acc_ref[1] = s_new

    @pl.when(i == NH - 1)
    def _():
        lse = m_new + jnp.log(s_new)
        out_ref[...] = out_ref[...] - lse


def _mk_w_spec(k):
    return pl.BlockSpec((BLK, D), lambda i, k=k: (i + k * NH, 0))


def _mk_b_spec(k):
    return pl.BlockSpec((1, BLK), lambda i, k=k: (0, i + k * NH))


_tc_matvec_lse = pl.pallas_call(
    _tc1_body,
    grid=(NH,),
    in_specs=(
        [pl.BlockSpec((1, D), lambda i: (0, 0))]
        + [_mk_w_spec(k) for k in range(NS)]
        + [_mk_b_spec(k) for k in range(NS)]
    ),
    out_specs=pl.BlockSpec((1, V), lambda i: (0, 0)),
    out_shape=jax.ShapeDtypeStruct((1, V), jnp.float32),
    scratch_shapes=[
        pltpu.SMEM((2,), jnp.float32),
    ],
    compiler_params=pltpu.CompilerParams(
        dimension_semantics=("arbitrary",)
    ),
)


def kernel(context, emb_table, W, b):
    context = context.astype(jnp.int32)
    v = _sc_gather_mean()(context, emb_table)
    b2 = b.reshape(1, V)
    return _tc_matvec_lse(v, *([W] * NS), *([b2] * NS))


# speedup vs baseline: 3.5024x; 1.6138x over previous
import functools

import jax
import jax.numpy as jnp
from jax import lax
from jax.experimental import pallas as pl
from jax.experimental.pallas import tpu as pltpu

V = 100000
D = 128
L = 200

BLK = 3200
NS = 4
NB = -(-V // BLK)
NH = NB // NS
VT = V - (NB - 1) * BLK
assert NB % NS == 0


def _tc_body(ctx_ref, emb_ref, *refs):
    w_refs = refs[0:NS]
    b_refs = refs[NS : 2 * NS]
    out_ref = refs[2 * NS]
    rows_ref = refs[2 * NS + 1]
    vm_ref = refs[2 * NS + 2]
    acc_ref = refs[2 * NS + 3]
    sem = refs[2 * NS + 4]
    i = pl.program_id(0)

    @pl.when(i == 0)
    def _():
        acc_ref[0] = -jnp.inf
        acc_ref[1] = 0.0
        copies = [
            pltpu.make_async_copy(
                emb_ref.at[pl.ds(ctx_ref[k], 1), :],
                rows_ref.at[pl.ds(k, 1), :],
                sem,
            )
            for k in range(L)
        ]
        for c in copies:
            c.start()
        for c in copies:
            c.wait()
        vsum = jnp.sum(rows_ref[...], axis=0, keepdims=True) * (1.0 / L)
        vm_ref[...] = lax.transpose(jnp.broadcast_to(vsum, (8, D)), (1, 0))

    vm = vm_ref[...]
    dn = (((1,), (0,)), ((), ()))
    los, valids, bmaxs = [], [], []
    for k in range(NS):
        lo_col = lax.dot_general(
            w_refs[k][...], vm, dn, preferred_element_type=jnp.float32
        )
        lo = lax.transpose(lo_col, (1, 0))[0:1, :]
        lo = lo + b_refs[k][...]
        base = (i + k * NH) * BLK
        if k == NS - 1:
            @pl.when(i < NH - 1)
            def _(lo=lo, base=base):
                out_ref[:, pl.ds(base, BLK)] = lo

            @pl.when(i == NH - 1)
            def _(lo=lo, base=base):
                out_ref[:, pl.ds(base, VT)] = lo[:, :VT]
        else:
            out_ref[:, pl.ds(base, BLK)] = lo
        pos = lax.broadcasted_iota(jnp.int32, (1, BLK), 1) + base
        valid = pos < V
        los.append(lo)
        valids.append(valid)
        bmaxs.append(jnp.max(jnp.where(valid, lo, -jnp.inf)))

    m_old = acc_ref[0]
    s_old = acc_ref[1]
    m_new = m_old
    for bm in bmaxs:
        m_new = jnp.maximum(m_new, bm)
    s_new = s_old * jnp.exp(m_old - m_new)
    for lo, valid in zip(los, valids):
        s_new = s_new + jnp.sum(jnp.where(valid, jnp.exp(lo - m_new), 0.0))
    acc_ref[0] = m_new
    acc_ref[1] = s_new

    @pl.when(i == NH - 1)
    def _():
        lse = m_new + jnp.log(s_new)
        out_ref[...] = out_ref[...] - lse


def _mk_w_spec(k):
    return pl.BlockSpec((BLK, D), lambda i, k=k: (i + k * NH, 0))


def _mk_b_spec(k):
    return pl.BlockSpec((1, BLK), lambda i, k=k: (0, i + k * NH))


_tc_cbow = pl.pallas_call(
    _tc_body,
    grid=(NH,),
    in_specs=(
        [
            pl.BlockSpec(memory_space=pltpu.SMEM),
            pl.BlockSpec(memory_space=pl.ANY),
        ]
        + [_mk_w_spec(k) for k in range(NS)]
        + [_mk_b_spec(k) for k in range(NS)]
    ),
    out_specs=pl.BlockSpec((1, V), lambda i: (0, 0)),
    out_shape=jax.ShapeDtypeStruct((1, V), jnp.float32),
    scratch_shapes=[
        pltpu.VMEM((L, D), jnp.float32),
        pltpu.VMEM((D, 8), jnp.float32),
        pltpu.SMEM((2,), jnp.float32),
        pltpu.SemaphoreType.DMA,
    ],
    compiler_params=pltpu.CompilerParams(
        dimension_semantics=("arbitrary",)
    ),
)


def kernel(context, emb_table, W, b):
    context = context.astype(jnp.int32)
    b2 = b.reshape(1, V)
    return _tc_cbow(context, emb_table, *([W] * NS), *([b2] * NS))


# speedup vs baseline: 3.6999x; 1.0564x over previous
import functools

import jax
import jax.numpy as jnp
from jax import lax
from jax.experimental import pallas as pl
from jax.experimental.pallas import tpu as pltpu

V = 100000
D = 128
L = 200

BLK = 3200
NS = 4
NB = -(-V // BLK)
NH = NB // NS
VT = V - (NB - 1) * BLK
assert NB % NS == 0


def _tc_body(ctx_ref, emb_ref, *refs):
    w_refs = refs[0:NS]
    b_refs = refs[NS : 2 * NS]
    out_ref = refs[2 * NS]
    rows_ref = refs[2 * NS + 1]
    vm_ref = refs[2 * NS + 2]
    sem = refs[2 * NS + 3]
    i = pl.program_id(0)

    @pl.when(i == 0)
    def _():
        copies = [
            pltpu.make_async_copy(
                emb_ref.at[pl.ds(ctx_ref[k], 1), :],
                rows_ref.at[pl.ds(k, 1), :],
                sem,
            )
            for k in range(L)
        ]
        for c in copies:
            c.start()
        for c in copies:
            c.wait()
        vsum = jnp.sum(rows_ref[...], axis=0, keepdims=True) * (1.0 / L)
        vm_ref[...] = lax.transpose(jnp.broadcast_to(vsum, (8, D)), (1, 0))

    vm = vm_ref[...]
    dn = (((1,), (0,)), ((), ()))
    for k in range(NS):
        lo_col = lax.dot_general(
            w_refs[k][...], vm, dn, preferred_element_type=jnp.float32
        )
        lo = lax.transpose(lo_col, (1, 0))[0:1, :]
        lo = lo + b_refs[k][...]
        base = (i + k * NH) * BLK
        if k == NS - 1:
            @pl.when(i < NH - 1)
            def _(lo=lo, base=base):
                out_ref[:, pl.ds(base, BLK)] = lo

            @pl.when(i == NH - 1)
            def _(lo=lo, base=base):
                out_ref[:, pl.ds(base, VT)] = lo[:, :VT]
        else:
            out_ref[:, pl.ds(base, BLK)] = lo

    @pl.when(i == NH - 1)
    def _():
        x = out_ref[...]
        m = jnp.max(x)
        lse = m + jnp.log(jnp.sum(jnp.exp(x - m)))
        out_ref[...] = x - lse


def _mk_w_spec(k):
    return pl.BlockSpec((BLK, D), lambda i, k=k: (i + k * NH, 0))


def _mk_b_spec(k):
    return pl.BlockSpec((1, BLK), lambda i, k=k: (0, i + k * NH))


_tc_cbow = pl.pallas_call(
    _tc_body,
    grid=(NH,),
    in_specs=(
        [
            pl.BlockSpec(memory_space=pltpu.SMEM),
            pl.BlockSpec(memory_space=pl.ANY),
        ]
        + [_mk_w_spec(k) for k in range(NS)]
        + [_mk_b_spec(k) for k in range(NS)]
    ),
    out_specs=pl.BlockSpec((1, V), lambda i: (0, 0)),
    out_shape=jax.ShapeDtypeStruct((1, V), jnp.float32),
    scratch_shapes=[
        pltpu.VMEM((L, D), jnp.float32),
        pltpu.VMEM((D, 8), jnp.float32),
        pltpu.SemaphoreType.DMA,
    ],
    compiler_params=pltpu.CompilerParams(
        dimension_semantics=("arbitrary",)
    ),
)


def kernel(context, emb_table, W, b):
    context = context.astype(jnp.int32)
    b2 = b.reshape(1, V)
    return _tc_cbow(context, emb_table, *([W] * NS), *([b2] * NS))


# speedup vs baseline: 3.9598x; 1.0703x over previous
import functools

import jax
import jax.numpy as jnp
from jax import lax
from jax.experimental import pallas as pl
from jax.experimental.pallas import tpu as pltpu

V = 100000
D = 128
L = 200

BLK = 3200
NS = 4
NB = -(-V // BLK)
NH = NB // NS
VT = V - (NB - 1) * BLK
assert NB % NS == 0


def _tc_body(ctx_ref, emb_ref, b_ref, *refs):
    w_refs = refs[0:NS]
    out_ref = refs[NS]
    rows_ref = refs[NS + 1]
    vm_ref = refs[NS + 2]
    sem = refs[NS + 3]
    i = pl.program_id(0)

    @pl.when(i == 0)
    def _():
        copies = [
            pltpu.make_async_copy(
                emb_ref.at[pl.ds(ctx_ref[k], 1), :],
                rows_ref.at[pl.ds(k, 1), :],
                sem,
            )
            for k in range(L)
        ]
        for c in copies:
            c.start()
        for c in copies:
            c.wait()
        vsum = jnp.sum(rows_ref[...], axis=0, keepdims=True) * (1.0 / L)
        vm_ref[...] = lax.transpose(jnp.broadcast_to(vsum, (8, D)), (1, 0))

    vm = vm_ref[...]
    dn = (((1,), (0,)), ((), ()))
    for k in range(NS):
        lo_col = lax.dot_general(
            w_refs[k][...], vm, dn, preferred_element_type=jnp.float32
        )
        base = (i + k * NH) * BLK
        lo = lax.transpose(lo_col, (1, 0))[0:1, :]
        lo = lo + b_ref[pl.ds(base, BLK)].reshape(1, BLK)
        if k == NS - 1:
            @pl.when(i < NH - 1)
            def _(lo=lo, base=base):
                out_ref[:, pl.ds(base, BLK)] = lo

            @pl.when(i == NH - 1)
            def _(lo=lo, base=base):
                out_ref[:, pl.ds(base, VT)] = lo[:, :VT]
        else:
            out_ref[:, pl.ds(base, BLK)] = lo

    @pl.when(i == NH - 1)
    def _():
        x = out_ref[...]
        m = jnp.max(x)
        lse = m + jnp.log(jnp.sum(jnp.exp(x - m)))
        out_ref[...] = x - lse


def _mk_w_spec(k):
    return pl.BlockSpec((BLK, D), lambda i, k=k: (i + k * NH, 0))


_tc_cbow = pl.pallas_call(
    _tc_body,
    grid=(NH,),
    in_specs=(
        [
            pl.BlockSpec(memory_space=pltpu.SMEM),
            pl.BlockSpec(memory_space=pl.ANY),
            pl.BlockSpec((V,), lambda i: (0,)),
        ]
        + [_mk_w_spec(k) for k in range(NS)]
    ),
    out_specs=pl.BlockSpec((1, V), lambda i: (0, 0)),
    out_shape=jax.ShapeDtypeStruct((1, V), jnp.float32),
    scratch_shapes=[
        pltpu.VMEM((L, D), jnp.float32),
        pltpu.VMEM((D, 8), jnp.float32),
        pltpu.SemaphoreType.DMA,
    ],
    compiler_params=pltpu.CompilerParams(
        dimension_semantics=("arbitrary",)
    ),
)


def kernel(context, emb_table, W, b):
    context = context.astype(jnp.int32)
    return _tc_cbow(context, emb_table, b, *([W] * NS))
